# Initial kernel scaffold; baseline (speedup 1.0000x reference)
import jax
import jax.numpy as jnp
from jax.experimental import pallas as pl


def kernel(retrieved_shifts, query_residue_code, retrieved_residue_codes, retrieved_shift_masks, rc_table):
    raise NotImplementedError("write your pallas kernel here")


# speedup vs baseline: 5.9289x; 5.9289x over previous
import dataclasses
import functools

import jax
import jax.numpy as jnp
from jax import lax
from jax.experimental import pallas as pl
from jax.experimental.pallas import tpu as pltpu
from jax.experimental.pallas import tpu_sc as plsc

NL = 16
ROWS_PER_BLK = 8


def _sc_correct(shifts2d, q2d, r2d, mwords, rcpad, n_rows):
    B, KS = shifts2d.shape
    K = r2d.shape[1]
    S = KS // K
    W = KS // 4
    NPAIR = n_rows * n_rows
    NPAIR_PAD = ((NPAIR + 1) // 2) * 2 + 6
    D2LEN = NPAIR_PAD * 8
    GRID = B // ROWS_PER_BLK
    GROUPS = KS // (3 * NL)

    mesh = plsc.VectorSubcoreMesh(core_axis_name="c", subcore_axis_name="s")
    cp = pltpu.CompilerParams()
    if "needs_layout_passes" in pltpu.CompilerParams.__dataclass_fields__:
        cp = dataclasses.replace(cp, needs_layout_passes=False)

    @functools.partial(
        pl.kernel,
        mesh=mesh,
        compiler_params=cp,
        out_type=jax.ShapeDtypeStruct((B, KS), jnp.float32),
        scratch_types=[
            pltpu.VMEM((128,), jnp.float32),
            pltpu.VMEM((D2LEN,), jnp.float32),
        ],
    )
    def sc(shifts_hbm, q_hbm, r_hbm, m_hbm, rc_hbm, out_hbm, rc_v, d2_v):
        pltpu.sync_copy(rc_hbm, rc_v)

        lane = lax.iota(jnp.int32, NL)

        pb = lane >> 3
        jv = lane & 7
        jmask = jv < S

        @pl.loop(0, D2LEN // NL)
        def _build(c):
            pairv = jnp.minimum(2 * c + pb, NPAIR - 1)
            qiv = pairv // n_rows
            riv = pairv - qiv * n_rows
            aq = plsc.load_gather(rc_v, [qiv * S + jv])
            ar = plsc.load_gather(rc_v, [riv * S + jv])
            valid = (aq == aq) & (ar == ar) & jmask
            d2_v[pl.ds(c * NL, NL)] = jnp.where(valid, aq - ar, 0.0)

        PP = [(lane + 16 * t) // S for t in range(3)]
        SP = [(lane + 16 * t) % S for t in range(3)]
        BITMASK = 1 << ((lane & 3) << 3)
        WPAT = lane >> 2

        def body(sh_v, q_v, r_v, m_v, o_v):
            @pl.loop(0, ROWS_PER_BLK)
            def _row(row):
                rowv = jnp.full((NL,), row, dtype=jnp.int32)
                qv = plsc.load_gather(q_v, [rowv, jnp.zeros((NL,), jnp.int32)])
                qbase = jnp.clip(qv, 0, n_rows - 1) * (n_rows * 8)

                @pl.loop(0, GROUPS)
                def _grp(g):
                    for t in range(3):
                        f0 = g * 48 + t * 16
                        pidx = PP[t] + 8 * g
                        rv = plsc.load_gather(r_v, [rowv, pidx])
                        rv = jnp.clip(rv, 0, n_rows - 1)
                        gidx = (rv << 3) + (SP[t] + qbase)
                        dv = plsc.load_gather(d2_v, [gidx])
                        sv = sh_v[row, pl.ds(f0, NL)]
                        wv = plsc.load_gather(m_v, [rowv, WPAT + (f0 >> 2)])
                        mb = (wv & BITMASK) != 0
                        o_v[row, pl.ds(f0, NL)] = jnp.where(mb, sv + dv, sv)

        pltpu.emit_pipeline(
            body,
            grid=(GRID,),
            in_specs=[
                pl.BlockSpec((ROWS_PER_BLK, KS), lambda i: (i, 0)),
                pl.BlockSpec((ROWS_PER_BLK, 1), lambda i: (i, 0)),
                pl.BlockSpec((ROWS_PER_BLK, K), lambda i: (i, 0)),
                pl.BlockSpec((ROWS_PER_BLK, W), lambda i: (i, 0)),
            ],
            out_specs=[pl.BlockSpec((ROWS_PER_BLK, KS), lambda i: (i, 0))],
            core_axis_name=("c", "s"),
            dimension_semantics=(pltpu.PARALLEL,),
        )(shifts_hbm, q_hbm, r_hbm, m_hbm, out_hbm)

    return sc(shifts2d, q2d, r2d, mwords, rcpad)


def kernel(retrieved_shifts, query_residue_code, retrieved_residue_codes,
           retrieved_shift_masks, rc_table):
    B, K, S = retrieved_shifts.shape
    n_rows = rc_table.shape[0]
    KS = K * S

    shifts2d = retrieved_shifts.reshape(B, KS)
    q2d = query_residue_code.astype(jnp.int32).reshape(B, 1)
    r2d = retrieved_residue_codes.astype(jnp.int32)
    mwords = retrieved_shift_masks.reshape(B, KS).view(jnp.int32)
    rcpad = jnp.pad(rc_table.reshape(-1), (0, 128 - n_rows * S))

    out2d = _sc_correct(shifts2d, q2d, r2d, mwords, rcpad, n_rows)
    return out2d.reshape(B, K, S)
